# baseline (speedup 1.0000x reference)
import math

import jax
import jax.numpy as jnp
import numpy as np
from jax.experimental import pallas as pl
from jax.experimental.pallas import tpu as pltpu

_ORDER = 13
_SEED = 48879

_CHUNKS = [64, 64, 128, 256, 512] + [1024] * 14 + [512, 256, 128, 64, 64]
_MAXCHUNK = 1024
_NBUF = 8
_MBUF = 4


def _hilbert_encode_vec(x, y, order):
    x = x.astype(np.int64).copy()
    y = y.astype(np.int64).copy()
    d = np.zeros_like(x)
    s = 1 << (order - 1)
    while s > 0:
        rx = ((x & s) > 0).astype(np.int64)
        ry = ((y & s) > 0).astype(np.int64)
        d += s * s * ((3 * rx) ^ ry)
        swap = ry == 0
        flip = swap & (rx == 1)
        x_f = np.where(flip, s - 1 - x, x)
        y_f = np.where(flip, s - 1 - y, y)
        x_new = np.where(swap, y_f, x_f)
        y_new = np.where(swap, x_f, y_f)
        x, y = x_new, y_new
        s >>= 1
    return d


def _v_mask_generative(addr_u64, rounds, seed):
    h = addr_u64 ^ np.uint64(seed & 0xFFFFFFFFFFFFFFFF)
    for _ in range(rounds):
        h = h * np.uint64(6364136223846793005) + np.uint64(1442695040888963407)
        h = h ^ (h >> np.uint64(33))
    frac = (h & np.uint64(0xFFFFFF)).astype(np.float64) / float(0xFFFFFF)
    return (frac * 2.0 - 1.0).astype(np.float32)


def _resonance_vec(S, D, seed_val, order):
    i = np.arange(S, dtype=np.int64)
    j = i.copy()
    t = _hilbert_encode_vec(i, j, order)
    addr = (i.astype(np.uint64) << np.uint64(32)) | j.astype(np.uint64)
    s_long = int(round(seed_val))
    ground_weight = _v_mask_generative(addr, 4, s_long ^ D)
    sig = (np.uint64(s_long) ^ np.uint64(D) ^ t.astype(np.uint64)) & np.uint64(0xFFFFFFFF)
    phase = (sig % np.uint64(1000)).astype(np.float64) / 1000.0 * 2.0 * math.pi
    resonance = np.sin(phase).astype(np.float32)
    return ground_weight * resonance


def _pipe_body(scale_ref, res_ref, q_hbm, o_hbm, inbuf, outbuf, in_sems, out_sems):
    s = scale_ref[0, 0]
    n_steps = len(_CHUNKS)
    offs = [0]
    for c in _CHUNKS:
        offs.append(offs[-1] + c)

    def in_copy(i, slot):
        c = _CHUNKS[i]
        return pltpu.make_async_copy(
            q_hbm.at[pl.ds(offs[i], c), :],
            inbuf.at[slot, pl.ds(0, c), :],
            in_sems.at[slot],
        )

    def out_copy(i, slot):
        c = _CHUNKS[i]
        return pltpu.make_async_copy(
            outbuf.at[slot, pl.ds(0, c), :],
            o_hbm.at[pl.ds(offs[i], c), :],
            out_sems.at[slot],
        )

    for j in range(min(_NBUF, n_steps)):
        in_copy(j, j).start()

    for i in range(n_steps):
        islot = i % _NBUF
        oslot = i % _MBUF
        if i >= _MBUF:
            out_copy(i - _MBUF, oslot).wait()
        in_copy(i, islot).wait()
        c = _CHUNKS[i]
        r = res_ref[pl.ds(offs[i], c), :]
        outbuf[oslot, pl.ds(0, c), :] = inbuf[islot, pl.ds(0, c), :] * (r * s)
        nxt = i + _NBUF
        if nxt < n_steps:
            in_copy(nxt, islot).start()
        out_copy(i, oslot).start()

    for i in range(max(0, n_steps - _MBUF), n_steps):
        out_copy(i, i % _MBUF).wait()


def kernel(q, k, v_val, latent_seed):
    B, S, D = q.shape
    res = _resonance_vec(S, D, float(_SEED), _ORDER)
    res_full = jnp.asarray(np.tile(res, B).reshape(B * S, 1))
    scale = (latent_seed * jnp.float32(1.0 / _SEED)).reshape(1, 1)

    rows = B * S
    q2 = q.reshape(rows, D)

    out = pl.pallas_call(
        _pipe_body,
        in_specs=[
            pl.BlockSpec(memory_space=pltpu.SMEM),
            pl.BlockSpec(memory_space=pltpu.VMEM),
            pl.BlockSpec(memory_space=pltpu.HBM),
        ],
        out_specs=pl.BlockSpec(memory_space=pltpu.HBM),
        out_shape=jax.ShapeDtypeStruct((rows, D), jnp.float32),
        scratch_shapes=[
            pltpu.VMEM((_NBUF, _MAXCHUNK, D), jnp.float32),
            pltpu.VMEM((_MBUF, _MAXCHUNK, D), jnp.float32),
            pltpu.SemaphoreType.DMA((_NBUF,)),
            pltpu.SemaphoreType.DMA((_MBUF,)),
        ],
    )(scale, res_full, q2)
    return out.reshape(B, S, D)


# speedup vs baseline: 1.0042x; 1.0042x over previous
import math

import jax
import jax.numpy as jnp
import numpy as np
from jax.experimental import pallas as pl
from jax.experimental.pallas import tpu as pltpu

_ORDER = 13
_SEED = 48879

_MAXCHUNK = 1024
_NBUF = 6
_MBUF = 6


def _chunk_schedule(rows):
    ramp = [64, 64, 128, 256, 512]
    edge = sum(ramp)
    if rows >= 4 * edge and rows % edge == 0:
        steady = rows - 2 * edge
        chunks = ramp + [_MAXCHUNK] * (steady // _MAXCHUNK)
        rem = steady % _MAXCHUNK
        if rem:
            chunks.append(rem)
        return chunks + ramp[::-1]
    chunks = []
    left = rows
    while left > 0:
        c = min(_MAXCHUNK, left)
        chunks.append(c)
        left -= c
    return chunks


def _hilbert_encode_vec(x, y, order):
    x = x.astype(np.int64).copy()
    y = y.astype(np.int64).copy()
    d = np.zeros_like(x)
    s = 1 << (order - 1)
    while s > 0:
        rx = ((x & s) > 0).astype(np.int64)
        ry = ((y & s) > 0).astype(np.int64)
        d += s * s * ((3 * rx) ^ ry)
        swap = ry == 0
        flip = swap & (rx == 1)
        x_f = np.where(flip, s - 1 - x, x)
        y_f = np.where(flip, s - 1 - y, y)
        x_new = np.where(swap, y_f, x_f)
        y_new = np.where(swap, x_f, y_f)
        x, y = x_new, y_new
        s >>= 1
    return d


def _v_mask_generative(addr_u64, rounds, seed):
    h = addr_u64 ^ np.uint64(seed & 0xFFFFFFFFFFFFFFFF)
    for _ in range(rounds):
        h = h * np.uint64(6364136223846793005) + np.uint64(1442695040888963407)
        h = h ^ (h >> np.uint64(33))
    frac = (h & np.uint64(0xFFFFFF)).astype(np.float64) / float(0xFFFFFF)
    return (frac * 2.0 - 1.0).astype(np.float32)


def _resonance_vec(S, D, seed_val, order):
    i = np.arange(S, dtype=np.int64)
    j = i.copy()
    t = _hilbert_encode_vec(i, j, order)
    addr = (i.astype(np.uint64) << np.uint64(32)) | j.astype(np.uint64)
    s_long = int(round(seed_val))
    ground_weight = _v_mask_generative(addr, 4, s_long ^ D)
    sig = (np.uint64(s_long) ^ np.uint64(D) ^ t.astype(np.uint64)) & np.uint64(0xFFFFFFFF)
    phase = (sig % np.uint64(1000)).astype(np.float64) / 1000.0 * 2.0 * math.pi
    resonance = np.sin(phase).astype(np.float32)
    return ground_weight * resonance


def _make_pipe_body(chunks):
    n_steps = len(chunks)
    offs = [0]
    for c in chunks:
        offs.append(offs[-1] + c)

    def _pipe_body(scale_ref, res_ref, q_hbm, o_hbm, inbuf, outbuf, in_sems, out_sems):
        s = scale_ref[0, 0]

        def in_copy(i, slot):
            c = chunks[i]
            return pltpu.make_async_copy(
                q_hbm.at[pl.ds(offs[i], c), :],
                inbuf.at[slot, pl.ds(0, c), :],
                in_sems.at[slot],
            )

        def out_copy(i, slot):
            c = chunks[i]
            return pltpu.make_async_copy(
                outbuf.at[slot, pl.ds(0, c), :],
                o_hbm.at[pl.ds(offs[i], c), :],
                out_sems.at[slot],
            )

        for j in range(min(_NBUF, n_steps)):
            in_copy(j, j).start()

        for i in range(n_steps):
            islot = i % _NBUF
            oslot = i % _MBUF
            if i >= _MBUF:
                out_copy(i - _MBUF, oslot).wait()
            in_copy(i, islot).wait()
            c = chunks[i]
            r = res_ref[pl.ds(offs[i], c), :]
            outbuf[oslot, pl.ds(0, c), :] = inbuf[islot, pl.ds(0, c), :] * (r * s)
            nxt = i + _NBUF
            if nxt < n_steps:
                in_copy(nxt, islot).start()
            out_copy(i, oslot).start()

        for i in range(max(0, n_steps - _MBUF), n_steps):
            out_copy(i, i % _MBUF).wait()

    return _pipe_body


def kernel(q, k, v_val, latent_seed):
    B, S, D = q.shape
    res = _resonance_vec(S, D, float(_SEED), _ORDER)
    res_full = jnp.asarray(np.tile(res, B).reshape(B * S, 1))
    scale = (latent_seed * jnp.float32(1.0 / _SEED)).reshape(1, 1)

    rows = B * S
    q2 = q.reshape(rows, D)
    chunks = _chunk_schedule(rows)

    out = pl.pallas_call(
        _make_pipe_body(chunks),
        in_specs=[
            pl.BlockSpec(memory_space=pltpu.SMEM),
            pl.BlockSpec(memory_space=pltpu.VMEM),
            pl.BlockSpec(memory_space=pltpu.HBM),
        ],
        out_specs=pl.BlockSpec(memory_space=pltpu.HBM),
        out_shape=jax.ShapeDtypeStruct((rows, D), jnp.float32),
        scratch_shapes=[
            pltpu.VMEM((_NBUF, _MAXCHUNK, D), jnp.float32),
            pltpu.VMEM((_MBUF, _MAXCHUNK, D), jnp.float32),
            pltpu.SemaphoreType.DMA((_NBUF,)),
            pltpu.SemaphoreType.DMA((_MBUF,)),
        ],
    )(scale, res_full, q2)
    return out.reshape(B, S, D)
